# baseline (speedup 1.0000x reference)
import functools

import jax
import jax.numpy as jnp
from jax import lax
from jax.experimental import pallas as pl
from jax.experimental.pallas import tpu as pltpu
from jax.experimental.pallas import tpu_sc as plsc

TOTAL_C = 256

NC = 2
NS = 16
NW = NC * NS

SLAB_B = 16
CHUNK = 8
ZCHUNK = 8


def _sc_scatter(x2, dst_idx, pad_idx, b, c_in, hw):
    n_pad_rows = b * (TOTAL_C - c_in)
    stream_b = b - SLAB_B
    rows_per_w = stream_b * c_in // NW
    prows_per_w = n_pad_rows // NW
    n_chunks = rows_per_w // CHUNK
    n_pchunks = prows_per_w // ZCHUNK
    slab_per_sc = SLAB_B // NC

    mesh = plsc.VectorSubcoreMesh(core_axis_name="c", subcore_axis_name="s")

    @functools.partial(
        pl.kernel,
        mesh=mesh,
        compiler_params=pltpu.CompilerParams(use_tc_tiling_on_sc=False),
        out_type=jax.ShapeDtypeStruct((b * TOTAL_C, hw), jnp.float32),
        scratch_types=[
            pltpu.VMEM((n_chunks, CHUNK), jnp.int32),
            pltpu.VMEM((n_pchunks, ZCHUNK), jnp.int32),
            pltpu.VMEM((CHUNK, hw), jnp.float32),
            pltpu.VMEM((CHUNK, hw), jnp.float32),
            pltpu.VMEM((ZCHUNK, hw), jnp.float32),
            pltpu.VMEM_SHARED((c_in // 2, hw), jnp.float32),
            pltpu.VMEM_SHARED((c_in // 2, hw), jnp.float32),
            pltpu.SemaphoreType.DMA,
            pltpu.SemaphoreType.DMA,
            pltpu.SemaphoreType.DMA,
            pltpu.SemaphoreType.DMA,
            pltpu.SemaphoreType.DMA,
            pltpu.SemaphoreType.DMA,
            pltpu.SemaphoreType.DMA,
            pltpu.SemaphoreType.DMA,
            pltpu.SemaphoreType.DMA,
        ],
    )
    def k(x_hbm, dsti_hbm, padi_hbm, out_hbm,
          idx_v, pidx_v, buf0, buf1, zbuf, slab0, slab1,
          gs0, gs1, ss0, ss1, zsem, lg0, lg1, lw0, lw1):
        cid = lax.axis_index("c")
        sid = lax.axis_index("s")
        wid = sid * NC + cid
        buf = (buf0, buf1)
        gsem = (gs0, gs1)
        ssem = (ss0, ss1)
        row0 = SLAB_B * c_in + wid * rows_per_w

        gh = {}
        gh[0] = pltpu.async_copy(
            x_hbm.at[pl.ds(row0, CHUNK)], buf[0], gsem[0])

        pltpu.sync_copy(dsti_hbm.at[wid], idx_v)
        pltpu.sync_copy(padi_hbm.at[wid], pidx_v)

        @pl.when(sid == 0)
        def _():
            slab = (slab0, slab1)
            lg = (lg0, lg1)
            lw = (lw0, lw1)
            b0 = cid * slab_per_sc
            half = c_in // 2
            n_steps = 2 * slab_per_sc

            def ssrc(i):
                return b0 * c_in + i * half

            def sdst(i):
                return (b0 + i // 2) * TOTAL_C + (i % 2) * half

            lgh = {}
            lwh = {}
            lgh[0] = pltpu.async_copy(
                x_hbm.at[pl.ds(ssrc(0), half)], slab[0], lg[0])
            for i in range(n_steps):
                cur = i & 1
                lgh[i].wait()
                lwh[i] = pltpu.async_copy(
                    slab[cur],
                    out_hbm.at[pl.ds(sdst(i), half)],
                    lw[cur],
                )
                if i + 1 < n_steps:
                    if i >= 1:
                        lwh[i - 1].wait()
                    lgh[i + 1] = pltpu.async_copy(
                        x_hbm.at[pl.ds(ssrc(i + 1), half)],
                        slab[1 - cur], lg[1 - cur])
            if n_steps >= 2:
                lwh[n_steps - 2].wait()
            lwh[n_steps - 1].wait()

        zvec = jnp.zeros((16,), jnp.float32)
        for i in range(ZCHUNK):

            def zfill(kk, carry, _i=i):
                zbuf[_i, pl.ds(kk * 16, 16)] = zvec
                return carry

            lax.fori_loop(0, hw // 16, zfill, 0)

        zh = [
            pltpu.async_copy(zbuf, out_hbm.at[pidx_v.at[j]], zsem)
            for j in range(n_pchunks)
        ]

        sh = {}
        for j in range(n_chunks):
            cur = j & 1
            gh[j].wait()
            sh[j] = pltpu.async_copy(
                buf[cur], out_hbm.at[idx_v.at[j]], ssem[cur])
            if j + 1 < n_chunks:
                if j >= 1:
                    sh[j - 1].wait()
                gh[j + 1] = pltpu.async_copy(
                    x_hbm.at[pl.ds(row0 + (j + 1) * CHUNK, CHUNK)],
                    buf[1 - cur], gsem[1 - cur])
        if n_chunks >= 2:
            sh[n_chunks - 2].wait()
        sh[n_chunks - 1].wait()
        for h in zh:
            h.wait()

    return k(x2, dst_idx, pad_idx)


def kernel(x, conv_forward_indices):
    b, c_in, h, w = x.shape
    hw = h * w
    idx = conv_forward_indices.astype(jnp.int32)

    base = jnp.arange(SLAB_B, b, dtype=jnp.int32)[:, None] * TOTAL_C
    dst_rows = (base + idx[None, :]).reshape(NW, -1, CHUNK)

    basez = jnp.arange(b, dtype=jnp.int32)[:, None] * TOTAL_C
    covered = jnp.zeros((TOTAL_C,), jnp.bool_).at[idx].set(True)
    pad_ch = jnp.nonzero(
        ~covered, size=TOTAL_C - c_in, fill_value=0)[0].astype(jnp.int32)
    pad_rows = (basez + pad_ch[None, :]).reshape(NW, -1, ZCHUNK)

    x2 = x.reshape(b * c_in, hw)
    out2 = _sc_scatter(x2, dst_rows, pad_rows, b, c_in, hw)
    return out2.reshape(b, TOTAL_C, h, w)


# speedup vs baseline: 1.0125x; 1.0125x over previous
import functools

import jax
import jax.numpy as jnp
from jax import lax
from jax.experimental import pallas as pl
from jax.experimental.pallas import tpu as pltpu
from jax.experimental.pallas import tpu_sc as plsc

TOTAL_C = 256

NC = 2
NS = 16
NW = NC * NS

CHUNK = 16
ZCHUNK = 8


def _sc_scatter(x2, dst_idx, pad_idx, n_rows, n_pad_rows, hw):
    rows_per_w = n_rows // NW
    prows_per_w = n_pad_rows // NW
    n_chunks = rows_per_w // CHUNK
    n_pchunks = prows_per_w // ZCHUNK

    mesh = plsc.VectorSubcoreMesh(core_axis_name="c", subcore_axis_name="s")

    @functools.partial(
        pl.kernel,
        mesh=mesh,
        compiler_params=pltpu.CompilerParams(use_tc_tiling_on_sc=False),
        out_type=jax.ShapeDtypeStruct((n_rows + n_pad_rows, hw), jnp.float32),
        scratch_types=[
            pltpu.VMEM((n_chunks, CHUNK), jnp.int32),
            pltpu.VMEM((n_pchunks, ZCHUNK), jnp.int32),
            pltpu.VMEM((CHUNK, hw), jnp.float32),
            pltpu.VMEM((CHUNK, hw), jnp.float32),
            pltpu.VMEM((ZCHUNK, hw), jnp.float32),
            pltpu.SemaphoreType.DMA,
            pltpu.SemaphoreType.DMA,
            pltpu.SemaphoreType.DMA,
            pltpu.SemaphoreType.DMA,
            pltpu.SemaphoreType.DMA,
        ],
    )
    def k(x_hbm, dsti_hbm, padi_hbm, out_hbm,
          idx_v, pidx_v, buf0, buf1, zbuf, gs0, gs1, ss0, ss1, zsem):
        wid = lax.axis_index("s") * NC + lax.axis_index("c")
        buf = (buf0, buf1)
        gsem = (gs0, gs1)
        ssem = (ss0, ss1)
        row0 = wid * rows_per_w

        gh = {}
        gh[0] = pltpu.async_copy(
            x_hbm.at[pl.ds(row0, CHUNK)], buf[0], gsem[0])

        pltpu.sync_copy(dsti_hbm.at[wid], idx_v)
        pltpu.sync_copy(padi_hbm.at[wid], pidx_v)

        zvec = jnp.zeros((16,), jnp.float32)
        for i in range(ZCHUNK):

            def zfill(kk, carry, _i=i):
                zbuf[_i, pl.ds(kk * 16, 16)] = zvec
                return carry

            lax.fori_loop(0, hw // 16, zfill, 0)

        zh = [
            pltpu.async_copy(zbuf, out_hbm.at[pidx_v.at[j]], zsem)
            for j in range(n_pchunks)
        ]

        sh = {}
        for j in range(n_chunks):
            cur = j & 1
            gh[j].wait()
            sh[j] = pltpu.async_copy(
                buf[cur], out_hbm.at[idx_v.at[j]], ssem[cur])
            if j + 1 < n_chunks:
                if j >= 1:
                    sh[j - 1].wait()
                gh[j + 1] = pltpu.async_copy(
                    x_hbm.at[pl.ds(row0 + (j + 1) * CHUNK, CHUNK)],
                    buf[1 - cur], gsem[1 - cur])
        if n_chunks >= 2:
            sh[n_chunks - 2].wait()
        sh[n_chunks - 1].wait()
        for h in zh:
            h.wait()

    return k(x2, dst_idx, pad_idx)


def kernel(x, conv_forward_indices):
    b, c_in, h, w = x.shape
    hw = h * w
    idx = conv_forward_indices.astype(jnp.int32)

    base = jnp.arange(b, dtype=jnp.int32)[:, None] * TOTAL_C
    dst_rows = (base + idx[None, :]).reshape(NW, -1, CHUNK)

    covered = jnp.zeros((TOTAL_C,), jnp.bool_).at[idx].set(True)
    pad_ch = jnp.nonzero(
        ~covered, size=TOTAL_C - c_in, fill_value=0)[0].astype(jnp.int32)
    pad_rows = (base + pad_ch[None, :]).reshape(NW, -1, ZCHUNK)

    x2 = x.reshape(b * c_in, hw)
    out2 = _sc_scatter(
        x2, dst_rows, pad_rows, b * c_in, b * (TOTAL_C - c_in), hw)
    return out2.reshape(b, TOTAL_C, h, w)
